# Initial kernel scaffold; baseline (speedup 1.0000x reference)
import jax
import jax.numpy as jnp
from jax.experimental import pallas as pl


def kernel(x, edge_index, W1_l, W1_r, b1, W2_l, W2_r, b2, W3, b3):
    raise NotImplementedError("write your pallas kernel here")


# speedup vs baseline: 10.0129x; 10.0129x over previous
import functools

import jax
import jax.numpy as jnp
from jax import lax
from jax.experimental import pallas as pl
from jax.experimental.pallas import tpu as pltpu
from jax.experimental.pallas import tpu_sc as plsc

N = 10000
E = 320000
NPAD = 10240
NC = 2
NS = 16
NW = NC * NS
E_PER_W = E // NW
CHUNK = 400
NCHUNK = E_PER_W // CHUNK
ROWS_PER_TILE = NPAD // NS
RB = 512
GRID = NPAD // RB
DEGW = 16


def _sc_mesh():
    return plsc.VectorSubcoreMesh(core_axis_name="c", subcore_axis_name="s",
                                  num_cores=NC, num_subcores=NS)


def _sc_params():
    return pltpu.CompilerParams(use_tc_tiling_on_sc=False)


def _make_sc_agg(d):
    out_type = [jax.ShapeDtypeStruct((NC * NPAD, d), jnp.float32)]
    scratch = [
        pltpu.VMEM((CHUNK,), jnp.int32),
        pltpu.VMEM((CHUNK,), jnp.int32),
        pltpu.VMEM((CHUNK, d), jnp.float32),
        pltpu.VMEM((ROWS_PER_TILE, d), jnp.float32),
        pltpu.VMEM_SHARED((NPAD, d), jnp.float32),
        pltpu.SemaphoreType.DMA,
    ]

    def body(y_hbm, src_hbm, dst_hbm, zrows_hbm, agg_hbm,
             sidx, didx, rows, stage, acc, sem):
        cid = lax.axis_index("c")
        sid = lax.axis_index("s")
        wid = cid * NS + sid
        my_rows = pl.ds(sid * ROWS_PER_TILE, ROWS_PER_TILE)

        pltpu.sync_copy(zrows_hbm, stage)
        pltpu.sync_copy(stage, acc.at[my_rows])
        plsc.subcore_barrier()

        ebase = wid * E_PER_W

        def chunk_step(c, carry):
            base = pl.multiple_of(ebase + c * CHUNK, 8)
            pltpu.sync_copy(src_hbm.at[pl.ds(base, CHUNK)], sidx)
            pltpu.sync_copy(dst_hbm.at[pl.ds(base, CHUNK)], didx)
            pltpu.async_copy(y_hbm.at[sidx], rows, sem).wait()
            pltpu.sync_copy(rows, acc.at[didx], add=True)
            return carry

        lax.fori_loop(0, NCHUNK, chunk_step, 0)
        plsc.subcore_barrier()

        obase = pl.multiple_of(cid * NPAD + sid * ROWS_PER_TILE, 8)
        pltpu.sync_copy(acc.at[my_rows], stage)
        pltpu.sync_copy(stage, agg_hbm.at[pl.ds(obase, ROWS_PER_TILE)])

    return pl.kernel(body, out_type=out_type, mesh=_sc_mesh(),
                     scratch_types=scratch, compiler_params=_sc_params())


def _make_sc_deg():
    out_type = [jax.ShapeDtypeStruct((NC * NPAD, DEGW), jnp.float32)]
    scratch = [
        pltpu.VMEM((CHUNK,), jnp.int32),
        pltpu.VMEM((CHUNK, DEGW), jnp.float32),
        pltpu.VMEM((ROWS_PER_TILE, DEGW), jnp.float32),
        pltpu.VMEM_SHARED((NPAD, DEGW), jnp.float32),
    ]

    def body(dst_hbm, zdeg_hbm, ones_hbm, deg_hbm,
             didx, onesb, dstage, dacc):
        cid = lax.axis_index("c")
        sid = lax.axis_index("s")
        wid = cid * NS + sid
        my_rows = pl.ds(sid * ROWS_PER_TILE, ROWS_PER_TILE)

        pltpu.sync_copy(zdeg_hbm, dstage)
        pltpu.sync_copy(dstage, dacc.at[my_rows])
        pltpu.sync_copy(ones_hbm, onesb)
        plsc.subcore_barrier()

        ebase = wid * E_PER_W

        def chunk_step(c, carry):
            base = pl.multiple_of(ebase + c * CHUNK, 8)
            pltpu.sync_copy(dst_hbm.at[pl.ds(base, CHUNK)], didx)
            pltpu.sync_copy(onesb, dacc.at[didx], add=True)
            return carry

        lax.fori_loop(0, NCHUNK, chunk_step, 0)
        plsc.subcore_barrier()

        obase = pl.multiple_of(cid * NPAD + sid * ROWS_PER_TILE, 8)
        pltpu.sync_copy(dacc.at[my_rows], dstage)
        pltpu.sync_copy(dstage, deg_hbm.at[pl.ds(obase, ROWS_PER_TILE)])

    return pl.kernel(body, out_type=out_type, mesh=_sc_mesh(),
                     scratch_types=scratch, compiler_params=_sc_params())


def _tc1(x_ref, wl_ref, wr_ref, b_ref, y1_ref, xr1_ref):
    xb = x_ref[...]
    y1_ref[...] = jnp.dot(xb, wl_ref[...], preferred_element_type=jnp.float32)
    xr1_ref[...] = (
        jnp.dot(xb, wr_ref[...], preferred_element_type=jnp.float32)
        + b_ref[...]
    )


def _tc2(a0_ref, a1_ref, d0_ref, d1_ref, xr_ref, wl_ref, wr_ref, b_ref,
         y2_ref, xr2_ref):
    agg = a0_ref[...] + a1_ref[...]
    deg = d0_ref[...][:, 0:1] + d1_ref[...][:, 0:1]
    h = jnp.maximum(agg / jnp.maximum(deg, 1.0) + xr_ref[...], 0.0)
    y2_ref[...] = jnp.dot(h, wl_ref[...], preferred_element_type=jnp.float32)
    xr2_ref[...] = (
        jnp.dot(h, wr_ref[...], preferred_element_type=jnp.float32)
        + b_ref[...]
    )


def _tc3(a0_ref, a1_ref, d0_ref, d1_ref, xr_ref, w3_ref, b3_ref, out_ref):
    agg = a0_ref[...] + a1_ref[...]
    deg = d0_ref[...][:, 0:1] + d1_ref[...][:, 0:1]
    h = jnp.maximum(agg / jnp.maximum(deg, 1.0) + xr_ref[...], 0.0)
    out = jnp.dot(h, w3_ref[...], preferred_element_type=jnp.float32) + b3_ref[...]
    out_ref[...] = jnp.maximum(out, 0.0)


def _row_spec(w):
    return pl.BlockSpec((RB, w), lambda i: (i, 0))


def _full_spec(shape):
    return pl.BlockSpec(shape, lambda i: tuple(0 for _ in shape))


def _split_specs(w):
    return (pl.BlockSpec((RB, w), lambda i: (i, 0)),
            pl.BlockSpec((RB, w), lambda i: (i + GRID, 0)))


def kernel(x, edge_index, W1_l, W1_r, b1, W2_l, W2_r, b2, W3, b3):
    src = edge_index[0]
    dst = edge_index[1]
    xpad = jnp.pad(x, ((0, NPAD - N), (0, 0)))

    zrows64 = jnp.zeros((ROWS_PER_TILE, 64), jnp.float32)
    zrows32 = jnp.zeros((ROWS_PER_TILE, 32), jnp.float32)
    zdeg = jnp.zeros((ROWS_PER_TILE, DEGW), jnp.float32)
    ones = jnp.ones((CHUNK, DEGW), jnp.float32)

    y1, xr1 = pl.pallas_call(
        _tc1,
        grid=(GRID,),
        in_specs=[_row_spec(128), _full_spec((128, 64)), _full_spec((128, 64)),
                  _full_spec((1, 64))],
        out_specs=[_row_spec(64), _row_spec(64)],
        out_shape=[jax.ShapeDtypeStruct((NPAD, 64), jnp.float32)] * 2,
    )(xpad, W1_l, W1_r, b1.reshape(1, 64))

    (agg1p,) = _make_sc_agg(64)(y1, src, dst, zrows64)
    (degp,) = _make_sc_deg()(dst, zdeg, ones)

    a_specs = _split_specs(64)
    d_specs = _split_specs(DEGW)
    y2, xr2 = pl.pallas_call(
        _tc2,
        grid=(GRID,),
        in_specs=[a_specs[0], a_specs[1], d_specs[0], d_specs[1],
                  _row_spec(64), _full_spec((64, 32)), _full_spec((64, 32)),
                  _full_spec((1, 32))],
        out_specs=[_row_spec(32), _row_spec(32)],
        out_shape=[jax.ShapeDtypeStruct((NPAD, 32), jnp.float32)] * 2,
    )(agg1p, agg1p, degp, degp, xr1, W2_l, W2_r, b2.reshape(1, 32))

    (agg2p,) = _make_sc_agg(32)(y2, src, dst, zrows32)

    a2_specs = _split_specs(32)
    out = pl.pallas_call(
        _tc3,
        grid=(GRID,),
        in_specs=[a2_specs[0], a2_specs[1], d_specs[0], d_specs[1],
                  _row_spec(32), _full_spec((32, 2)), _full_spec((1, 2))],
        out_specs=[_row_spec(2)],
        out_shape=[jax.ShapeDtypeStruct((NPAD, 2), jnp.float32)],
    )(agg2p, agg2p, degp, degp, xr2, W3, b3.reshape(1, 2))[0]

    return out[:N]


# speedup vs baseline: 13.6292x; 1.3612x over previous
import functools

import jax
import jax.numpy as jnp
from jax import lax
from jax.experimental import pallas as pl
from jax.experimental.pallas import tpu as pltpu
from jax.experimental.pallas import tpu_sc as plsc

N = 10000
E = 320000
NPAD = 10240
NC = 2
NS = 16
NW = NC * NS
E_PER_W = E // NW
CHUNK = 200
NCHUNK = E_PER_W // CHUNK
NPAIR = NCHUNK // 2
ROWS_PER_TILE = NPAD // NS
RB = 512
GRID = NPAD // RB
DEGW = 16


def _sc_mesh():
    return plsc.VectorSubcoreMesh(core_axis_name="c", subcore_axis_name="s",
                                  num_cores=NC, num_subcores=NS)


def _sc_params():
    return pltpu.CompilerParams(use_tc_tiling_on_sc=False)


def _make_sc_agg(d):
    out_type = [jax.ShapeDtypeStruct((NC * NPAD, d), jnp.float32)]
    scratch = [
        pltpu.VMEM((NCHUNK, CHUNK), jnp.int32),
        pltpu.VMEM((NCHUNK, CHUNK), jnp.int32),
        pltpu.VMEM((CHUNK, d), jnp.float32),
        pltpu.VMEM((CHUNK, d), jnp.float32),
        pltpu.VMEM((ROWS_PER_TILE, d), jnp.float32),
        pltpu.VMEM_SHARED((NPAD, d), jnp.float32),
        pltpu.SemaphoreType.DMA,
        pltpu.SemaphoreType.DMA,
    ]

    def body(y_hbm, src_hbm, dst_hbm, zrows_hbm, agg_hbm,
             sidx, didx, rows0, rows1, stage, acc, sem0, sem1):
        cid = lax.axis_index("c")
        sid = lax.axis_index("s")
        wid = cid * NS + sid
        my_rows = pl.ds(sid * ROWS_PER_TILE, ROWS_PER_TILE)

        pltpu.sync_copy(zrows_hbm, stage)
        pltpu.sync_copy(stage, acc.at[my_rows])
        cbase = pl.multiple_of(wid * NCHUNK, 8)
        pltpu.sync_copy(src_hbm.at[pl.ds(cbase, NCHUNK)], sidx)
        pltpu.sync_copy(dst_hbm.at[pl.ds(cbase, NCHUNK)], didx)
        plsc.subcore_barrier()

        def gather(c, buf, sem):
            pltpu.async_copy(y_hbm.at[sidx.at[c]], buf, sem)

        def gather_wait(c, buf, sem):
            pltpu.make_async_copy(y_hbm.at[sidx.at[c]], buf, sem).wait()

        gather(0, rows0, sem0)
        gather(1, rows1, sem1)

        def pair_step(i, carry):
            c0 = i * 2
            c1 = c0 + 1
            gather_wait(c0, rows0, sem0)
            pltpu.sync_copy(rows0, acc.at[didx.at[c0]], add=True)

            @pl.when(c0 + 2 < NCHUNK)
            def _():
                gather(c0 + 2, rows0, sem0)

            gather_wait(c1, rows1, sem1)
            pltpu.sync_copy(rows1, acc.at[didx.at[c1]], add=True)

            @pl.when(c1 + 2 < NCHUNK)
            def _():
                gather(c1 + 2, rows1, sem1)

            return carry

        lax.fori_loop(0, NPAIR, pair_step, 0)
        plsc.subcore_barrier()

        obase = pl.multiple_of(cid * NPAD + sid * ROWS_PER_TILE, 8)
        pltpu.sync_copy(acc.at[my_rows], stage)
        pltpu.sync_copy(stage, agg_hbm.at[pl.ds(obase, ROWS_PER_TILE)])

    return pl.kernel(body, out_type=out_type, mesh=_sc_mesh(),
                     scratch_types=scratch, compiler_params=_sc_params())


def _make_sc_deg():
    out_type = [jax.ShapeDtypeStruct((NC * NPAD, DEGW), jnp.float32)]
    scratch = [
        pltpu.VMEM((NCHUNK, CHUNK), jnp.int32),
        pltpu.VMEM((CHUNK, DEGW), jnp.float32),
        pltpu.VMEM((ROWS_PER_TILE, DEGW), jnp.float32),
        pltpu.VMEM_SHARED((NPAD, DEGW), jnp.float32),
        pltpu.SemaphoreType.DMA,
    ]

    def body(dst_hbm, zdeg_hbm, ones_hbm, deg_hbm,
             didx, onesb, dstage, dacc, sem):
        cid = lax.axis_index("c")
        sid = lax.axis_index("s")
        wid = cid * NS + sid
        my_rows = pl.ds(sid * ROWS_PER_TILE, ROWS_PER_TILE)

        pltpu.sync_copy(zdeg_hbm, dstage)
        pltpu.sync_copy(dstage, dacc.at[my_rows])
        pltpu.sync_copy(ones_hbm, onesb)
        cbase = pl.multiple_of(wid * NCHUNK, 8)
        pltpu.sync_copy(dst_hbm.at[pl.ds(cbase, NCHUNK)], didx)
        plsc.subcore_barrier()

        def fire(c, carry):
            pltpu.async_copy(onesb, dacc.at[didx.at[c]], sem, add=True)
            return carry

        def drain(c, carry):
            pltpu.make_async_copy(onesb, dacc.at[didx.at[c]], sem).wait()
            return carry

        lax.fori_loop(0, NCHUNK, fire, 0)
        lax.fori_loop(0, NCHUNK, drain, 0)
        plsc.subcore_barrier()

        obase = pl.multiple_of(cid * NPAD + sid * ROWS_PER_TILE, 8)
        pltpu.sync_copy(dacc.at[my_rows], dstage)
        pltpu.sync_copy(dstage, deg_hbm.at[pl.ds(obase, ROWS_PER_TILE)])

    return pl.kernel(body, out_type=out_type, mesh=_sc_mesh(),
                     scratch_types=scratch, compiler_params=_sc_params())


def _tc1(x_ref, wl_ref, wr_ref, b_ref, y1_ref, xr1_ref):
    xb = x_ref[...]
    y1_ref[...] = jnp.dot(xb, wl_ref[...], preferred_element_type=jnp.float32)
    xr1_ref[...] = (
        jnp.dot(xb, wr_ref[...], preferred_element_type=jnp.float32)
        + b_ref[...]
    )


def _tc2(a0_ref, a1_ref, d0_ref, d1_ref, xr_ref, wl_ref, wr_ref, b_ref,
         y2_ref, xr2_ref):
    agg = a0_ref[...] + a1_ref[...]
    deg = d0_ref[...][:, 0:1] + d1_ref[...][:, 0:1]
    h = jnp.maximum(agg / jnp.maximum(deg, 1.0) + xr_ref[...], 0.0)
    y2_ref[...] = jnp.dot(h, wl_ref[...], preferred_element_type=jnp.float32)
    xr2_ref[...] = (
        jnp.dot(h, wr_ref[...], preferred_element_type=jnp.float32)
        + b_ref[...]
    )


def _tc3(a0_ref, a1_ref, d0_ref, d1_ref, xr_ref, w3_ref, b3_ref, out_ref):
    agg = a0_ref[...] + a1_ref[...]
    deg = d0_ref[...][:, 0:1] + d1_ref[...][:, 0:1]
    h = jnp.maximum(agg / jnp.maximum(deg, 1.0) + xr_ref[...], 0.0)
    out = jnp.dot(h, w3_ref[...], preferred_element_type=jnp.float32) + b3_ref[...]
    out_ref[...] = jnp.maximum(out, 0.0)


def _row_spec(w):
    return pl.BlockSpec((RB, w), lambda i: (i, 0))


def _full_spec(shape):
    return pl.BlockSpec(shape, lambda i: tuple(0 for _ in shape))


def _split_specs(w):
    return (pl.BlockSpec((RB, w), lambda i: (i, 0)),
            pl.BlockSpec((RB, w), lambda i: (i + GRID, 0)))


def kernel(x, edge_index, W1_l, W1_r, b1, W2_l, W2_r, b2, W3, b3):
    src = edge_index[0].reshape(NW * NCHUNK, CHUNK)
    dst = edge_index[1].reshape(NW * NCHUNK, CHUNK)
    xpad = jnp.pad(x, ((0, NPAD - N), (0, 0)))

    zrows64 = jnp.zeros((ROWS_PER_TILE, 64), jnp.float32)
    zrows32 = jnp.zeros((ROWS_PER_TILE, 32), jnp.float32)
    zdeg = jnp.zeros((ROWS_PER_TILE, DEGW), jnp.float32)
    ones = jnp.ones((CHUNK, DEGW), jnp.float32)

    y1, xr1 = pl.pallas_call(
        _tc1,
        grid=(GRID,),
        in_specs=[_row_spec(128), _full_spec((128, 64)), _full_spec((128, 64)),
                  _full_spec((1, 64))],
        out_specs=[_row_spec(64), _row_spec(64)],
        out_shape=[jax.ShapeDtypeStruct((NPAD, 64), jnp.float32)] * 2,
    )(xpad, W1_l, W1_r, b1.reshape(1, 64))

    (agg1p,) = _make_sc_agg(64)(y1, src, dst, zrows64)
    (degp,) = _make_sc_deg()(dst, zdeg, ones)

    a_specs = _split_specs(64)
    d_specs = _split_specs(DEGW)
    y2, xr2 = pl.pallas_call(
        _tc2,
        grid=(GRID,),
        in_specs=[a_specs[0], a_specs[1], d_specs[0], d_specs[1],
                  _row_spec(64), _full_spec((64, 32)), _full_spec((64, 32)),
                  _full_spec((1, 32))],
        out_specs=[_row_spec(32), _row_spec(32)],
        out_shape=[jax.ShapeDtypeStruct((NPAD, 32), jnp.float32)] * 2,
    )(agg1p, agg1p, degp, degp, xr1, W2_l, W2_r, b2.reshape(1, 32))

    (agg2p,) = _make_sc_agg(32)(y2, src, dst, zrows32)

    a2_specs = _split_specs(32)
    out = pl.pallas_call(
        _tc3,
        grid=(GRID,),
        in_specs=[a2_specs[0], a2_specs[1], d_specs[0], d_specs[1],
                  _row_spec(32), _full_spec((32, 2)), _full_spec((1, 2))],
        out_specs=[_row_spec(2)],
        out_shape=[jax.ShapeDtypeStruct((NPAD, 2), jnp.float32)],
    )(agg2p, agg2p, degp, degp, xr2, W3, b3.reshape(1, 2))[0]

    return out[:N]
